# baseline (device time: 223830 ns/iter reference)
import jax
import jax.numpy as jnp
from jax import lax
from jax.experimental import pallas as pl
from jax.experimental.pallas import tpu as pltpu

N_DEV = 4
N_TOK = 4096
D_IN = 1024
D_OUT = 2048
HALF = D_OUT // 2
E_LOCAL = 4
CHUNK = N_TOK // N_DEV
N_STRIP = 4
RS_H = CHUNK // N_STRIP

W_SEQ = [0, 1, 2, 3] * (N_DEV + 2)

_DevIdType = getattr(pl, "DeviceIdType", None) or pltpu.DeviceIdType
_CompilerParams = getattr(pltpu, "CompilerParams", None) or getattr(
    pltpu, "TPUCompilerParams"
)
_sem_signal = getattr(pl, "semaphore_signal", None) or pltpu.semaphore_signal
_sem_wait = getattr(pl, "semaphore_wait", None) or pltpu.semaphore_wait
_ANY = pl.ANY
_VMEM_SPACE = pltpu.MemorySpace.VMEM
_MESH = _DevIdType.MESH


def _cast_w_bf16(expert_W):
    def body(in_ref, out_ref):
        out_ref[...] = in_ref[...].astype(jnp.bfloat16)

    return pl.pallas_call(
        body,
        grid=(E_LOCAL,),
        in_specs=[pl.BlockSpec((1, D_IN, D_OUT), lambda e: (e, 0, 0))],
        out_specs=pl.BlockSpec((1, D_IN, D_OUT), lambda e: (e, 0, 0)),
        out_shape=jax.ShapeDtypeStruct((E_LOCAL, D_IN, D_OUT), jnp.bfloat16),
    )(expert_W)


def _fused_moe_ar(x, route_idx, w_bf16):
    def body(
        route_ref,
        x_hbm,
        w_hbm,
        out_hbm,
        xbuf,
        wbuf,
        comm,
        work,
        fstage,
        xsems,
        wsems,
        fsems,
        send_r,
        recv_r,
        send_l,
        recv_l,
    ):
        my = lax.axis_index("i")
        left = lax.rem(my + N_DEV - 1, N_DEV)
        right = lax.rem(my + 1, N_DEV)

        offs = [
            lax.rem(my + d + N_DEV, N_DEV) * CHUNK for d in (0, -1, 1, 2)
        ]

        _COLR = slice(0, HALF)
        _COLL = slice(HALF, D_OUT)

        def x_copy(k):
            return pltpu.make_async_copy(
                x_hbm.at[pl.ds(offs[k], CHUNK), :],
                xbuf.at[k % 2],
                xsems.at[k % 2],
            )

        def w_copy(seq):
            return pltpu.make_async_copy(
                w_hbm.at[W_SEQ[seq]], wbuf.at[seq % 2], wsems.at[seq % 2]
            )

        def rs_rdma(s, rightward, t):
            if rightward:
                sc = lax.rem(my - s + N_DEV, N_DEV)
                cols, dev, ss, rs_ = _COLR, right, send_r, recv_r
            else:
                sc = lax.rem(my + s, N_DEV)
                cols, dev, ss, rs_ = _COLL, left, send_l, recv_l
            return pltpu.make_async_remote_copy(
                src_ref=work.at[pl.ds(sc * CHUNK + t * RS_H, RS_H), cols],
                dst_ref=comm.at[s, pl.ds(t * RS_H, RS_H), cols],
                send_sem=ss.at[s, t],
                recv_sem=rs_.at[s, t],
                device_id=(dev,),
                device_id_type=_MESH,
            )

        def ag_rdma(s, rightward, t):
            h = N_DEV - 1 + s
            if rightward:
                c = lax.rem(my + 1 - s + N_DEV, N_DEV)
                cols, dev, ss, rs_ = _COLR, right, send_r, recv_r
            else:
                c = lax.rem(my - 1 + s + N_DEV, N_DEV)
                cols, dev, ss, rs_ = _COLL, left, send_l, recv_l
            rows = pl.ds(c * CHUNK + t * RS_H, RS_H)
            return pltpu.make_async_remote_copy(
                src_ref=work.at[rows, cols],
                dst_ref=work.at[rows, cols],
                send_sem=ss.at[h, t],
                recv_sem=rs_.at[h, t],
                device_id=(dev,),
                device_id_type=_MESH,
            )

        def rs_add(s, k, cols, t):
            rows = pl.ds(offs[k] + t * RS_H, RS_H)
            crows = slice(t * RS_H, (t + 1) * RS_H)
            work[rows, cols] += comm[s, crows, cols]

        pending = [None, None]
        emit_n = [0]

        def emit(row_start, cols):
            slot = emit_n[0] % 2
            if pending[slot] is not None:
                pending[slot].wait()
            fstage[slot] = work[pl.ds(row_start, RS_H), cols].astype(
                jnp.float32
            )
            cp = pltpu.make_async_copy(
                fstage.at[slot],
                out_hbm.at[pl.ds(row_start, RS_H), cols],
                fsems.at[slot],
            )
            cp.start()
            pending[slot] = cp
            emit_n[0] += 1

        x_copy(0).start()
        w_copy(0).start()

        barrier = pltpu.get_barrier_semaphore()
        _sem_signal(barrier, inc=1, device_id=(left,), device_id_type=_MESH)
        _sem_signal(barrier, inc=1, device_id=(right,), device_id_type=_MESH)
        _sem_wait(barrier, 2)

        def compute_rows(k, r0, r1, seq0):
            rows = route_ref[pl.ds(offs[k] + r0, r1 - r0), :]
            orows = pl.ds(offs[k] + r0, r1 - r0)
            for i in range(E_LOCAL):
                seq = seq0 + i
                w_copy(seq).wait()
                if seq + 1 < len(W_SEQ):
                    w_copy(seq + 1).start()
                mask = rows == my * E_LOCAL + i
                xm = jnp.where(mask, xbuf[k % 2][r0:r1, :], 0.0).astype(
                    jnp.bfloat16
                )
                for h in range(2):
                    cols = slice(h * HALF, (h + 1) * HALF)
                    prod = jnp.dot(
                        xm,
                        wbuf[seq % 2][:, cols],
                        preferred_element_type=jnp.float32,
                    ).astype(jnp.bfloat16)
                    if i == 0:
                        work[orows, cols] = prod
                    else:
                        work[orows, cols] += prod

        x_copy(0).wait()
        x_copy(1).start()
        compute_rows(0, 0, CHUNK // 2, 0)
        for t in range(N_STRIP // 2):
            rs_rdma(0, True, t).start()
            rs_rdma(0, False, t).start()
        compute_rows(0, CHUNK // 2, CHUNK, E_LOCAL)
        for t in range(N_STRIP // 2, N_STRIP):
            rs_rdma(0, True, t).start()
            rs_rdma(0, False, t).start()

        def compute_chunk(k):
            x_copy(k).wait()
            if k < N_DEV - 1:
                x_copy(k + 1).start()
            compute_rows(k, 0, CHUNK, E_LOCAL * (k + 1))

        compute_chunk(1)
        for t in range(N_STRIP):
            rs_rdma(0, True, t).wait()
            rs_add(0, 1, _COLR, t)
            rs_rdma(1, True, t).start()

        compute_chunk(2)
        for t in range(N_STRIP):
            rs_rdma(0, False, t).wait()
            rs_add(0, 2, _COLL, t)
            rs_rdma(1, False, t).start()

        x_copy(3).wait()
        compute_rows(3, 0, CHUNK // 2, 4 * E_LOCAL)
        for t in range(N_STRIP // 2):
            rs_rdma(1, True, t).wait()
            rs_add(1, 3, _COLR, t)
            rs_rdma(2, True, t).start()
        for t in range(N_STRIP // 2):
            rs_rdma(1, False, t).wait()
            rs_add(1, 3, _COLL, t)
            rs_rdma(2, False, t).start()
        compute_rows(3, CHUNK // 2, CHUNK, 5 * E_LOCAL)
        for t in range(N_STRIP // 2, N_STRIP):
            rs_rdma(1, True, t).wait()
            rs_add(1, 3, _COLR, t)
            rs_rdma(2, True, t).start()
        for t in range(N_STRIP // 2, N_STRIP):
            rs_rdma(1, False, t).wait()
            rs_add(1, 3, _COLL, t)
            rs_rdma(2, False, t).start()
        for t in range(N_STRIP):
            rs_rdma(2, True, t).wait()
            rs_add(2, 2, _COLR, t)
            ag_rdma(0, True, t).start()
            emit(offs[2] + t * RS_H, _COLR)
        for t in range(N_STRIP):
            rs_rdma(2, False, t).wait()
            rs_add(2, 1, _COLL, t)
            ag_rdma(0, False, t).start()
            emit(offs[1] + t * RS_H, _COLL)

        for s in range(N_DEV - 1):
            cr = lax.rem(my - s + N_DEV, N_DEV) * CHUNK
            cl = lax.rem(my + s, N_DEV) * CHUNK
            for t in range(N_STRIP):
                ag_rdma(s, True, t).wait()
                if s < N_DEV - 2:
                    ag_rdma(s + 1, True, t).start()
                emit(cr + t * RS_H, _COLR)
            for t in range(N_STRIP):
                ag_rdma(s, False, t).wait()
                if s < N_DEV - 2:
                    ag_rdma(s + 1, False, t).start()
                emit(cl + t * RS_H, _COLL)

        for slot in range(2):
            if pending[slot] is not None:
                pending[slot].wait()

    n_hops = 2 * (N_DEV - 1)
    return pl.pallas_call(
        body,
        out_shape=jax.ShapeDtypeStruct((N_TOK, D_OUT), jnp.float32),
        in_specs=[
            pl.BlockSpec(memory_space=_VMEM_SPACE),
            pl.BlockSpec(memory_space=_ANY),
            pl.BlockSpec(memory_space=_ANY),
        ],
        out_specs=pl.BlockSpec(memory_space=_ANY),
        scratch_shapes=[
            pltpu.VMEM((2, CHUNK, D_IN), jnp.float32),
            pltpu.VMEM((2, D_IN, D_OUT), jnp.bfloat16),
            pltpu.VMEM((N_DEV - 1, CHUNK, D_OUT), jnp.bfloat16),
            pltpu.VMEM((N_TOK, D_OUT), jnp.bfloat16),
            pltpu.VMEM((2, RS_H, HALF), jnp.float32),
            pltpu.SemaphoreType.DMA((2,)),
            pltpu.SemaphoreType.DMA((2,)),
            pltpu.SemaphoreType.DMA((2,)),
            pltpu.SemaphoreType.DMA((n_hops, N_STRIP)),
            pltpu.SemaphoreType.DMA((n_hops, N_STRIP)),
            pltpu.SemaphoreType.DMA((n_hops, N_STRIP)),
            pltpu.SemaphoreType.DMA((n_hops, N_STRIP)),
        ],
        compiler_params=_CompilerParams(
            collective_id=0,
            vmem_limit_bytes=63 * 1024 * 1024,
        ),
    )(route_idx, x, w_bf16)


def kernel(x, router_W, route_idx, expert_W):
    del router_W
    w_bf16 = _cast_w_bf16(expert_W)
    return _fused_moe_ar(x, route_idx, w_bf16)
